# baseline (speedup 1.0000x reference)
import jax
import jax.numpy as jnp
from jax.experimental import pallas as pl


def _mm_kernel(w_ref, x_ref, o_ref):
    w = w_ref[...].astype(jnp.bfloat16)
    x = x_ref[...].astype(jnp.bfloat16)
    o_ref[...] = jax.lax.dot_general(
        w, x, (((0,), (0,)), ((), ())),
        preferred_element_type=jnp.float32)


def kernel(in_values, W, bias):
    x = in_values
    if x.ndim == 1:
        x = x.reshape(x.shape[0], 1)
    if x.shape[0] != W.shape[0]:
        x = x.T
    k, m = W.shape
    n = x.shape[1]
    bm = 512
    out = pl.pallas_call(
        _mm_kernel,
        grid=(m // bm,),
        in_specs=[
            pl.BlockSpec((k, bm), lambda i: (0, i)),
            pl.BlockSpec((k, n), lambda i: (0, 0)),
        ],
        out_specs=pl.BlockSpec((bm, n), lambda i: (i, 0)),
        out_shape=jax.ShapeDtypeStruct((m, n), jnp.float32),
    )(W, x)
    return out
